# baseline (speedup 1.0000x reference)
import functools

import jax
import jax.numpy as jnp
from jax import lax
from jax.experimental import pallas as pl
from jax.experimental.pallas import tpu as pltpu
from jax.experimental.pallas import tpu_sc as plsc

B = 16384
D = 8
NCOLS = 37
NC, NS, L = 2, 16, 16
NW = NC * NS
BPW = B // NW
CHUNK = 128
NCHUNK = BPW // CHUNK
GROUP = L
NGROUP = BPW // GROUP

_mesh = plsc.VectorSubcoreMesh(
    core_axis_name="c", subcore_axis_name="s", num_cores=NC, num_subcores=NS
)


@functools.partial(
    pl.kernel,
    out_type=jax.ShapeDtypeStruct((B, NCOLS), jnp.float32),
    mesh=_mesh,
    compiler_params=pltpu.CompilerParams(
        needs_layout_passes=False, use_tc_tiling_on_sc=False
    ),
    scratch_types=[
        pltpu.VMEM((BPW,), jnp.int32),
        pltpu.VMEM((BPW, D), jnp.float32),
        pltpu.VMEM((BPW, 7), jnp.float32),
        pltpu.VMEM((BPW, 21), jnp.float32),
        pltpu.VMEM((BPW,), jnp.int32),
        pltpu.VMEM((BPW, NCOLS), jnp.float32),
        pltpu.SemaphoreType.DMA,
    ],
)
def _encode(gender_hbm, age_hbm, occ_hbm, idx_hbm, table_hbm, out_hbm,
            idx_v, z_v, a_v, o_v, g_v, s_v, sem):
    wid = lax.axis_index("s") * NC + lax.axis_index("c")
    base = wid * BPW

    pltpu.sync_copy(idx_hbm.at[pl.ds(base, BPW)], idx_v)
    copies = []
    for j in range(NCHUNK):
        sl = pl.ds(j * CHUNK, CHUNK)
        copies.append(
            pltpu.async_copy(table_hbm.at[idx_v.at[sl]], z_v.at[sl], sem)
        )

    pltpu.sync_copy(age_hbm.at[pl.ds(base, BPW)], a_v)
    pltpu.sync_copy(occ_hbm.at[pl.ds(base, BPW)], o_v)
    pltpu.sync_copy(gender_hbm.at[pl.ds(base, BPW)], g_v)
    for c in copies:
        c.wait()

    lane = lax.iota(jnp.int32, L)

    def lanes(nvec, width):
        rows, cols = [], []
        for k in range(nvec):
            j = lane + k * L
            rows.append(lax.div(j, jnp.int32(width)))
            cols.append(lax.rem(j, jnp.int32(width)))
        return rows, cols

    a_r, a_c = lanes(7, 7)
    o_r, o_c = lanes(21, 21)
    z_r, z_c = lanes(8, 8)
    zero = jnp.zeros((L,), jnp.int32)

    def group_body(g, carry):
        grow = g * GROUP
        gvals = g_v[pl.ds(grow, L)].astype(jnp.float32)
        plsc.store_scatter(s_v, [lane + grow, zero], gvals)
        for k in range(7):
            vals = plsc.load_gather(a_v, [a_r[k] + grow, a_c[k]])
            plsc.store_scatter(s_v, [a_r[k] + grow, a_c[k] + 1], vals)
        for k in range(21):
            vals = plsc.load_gather(o_v, [o_r[k] + grow, o_c[k]])
            plsc.store_scatter(s_v, [o_r[k] + grow, o_c[k] + 8], vals)
        for k in range(8):
            vals = plsc.load_gather(z_v, [z_r[k] + grow, z_c[k]])
            plsc.store_scatter(s_v, [z_r[k] + grow, z_c[k] + 29], vals)
        return carry

    lax.fori_loop(0, NGROUP, group_body, 0)

    pltpu.sync_copy(s_v, out_hbm.at[pl.ds(base, BPW)])


def kernel(gender, age, occupation, zipcode_bucket, zipcode_table):
    return _encode(
        gender.astype(jnp.int32),
        age,
        occupation,
        zipcode_bucket.astype(jnp.int32),
        zipcode_table,
    )


# speedup vs baseline: 2.0551x; 2.0551x over previous
import functools

import jax
import jax.numpy as jnp
from jax import lax
from jax.experimental import pallas as pl
from jax.experimental.pallas import tpu as pltpu
from jax.experimental.pallas import tpu_sc as plsc

B = 16384
V = 100000
D = 8
NCOLS = 37
NC, NS, L = 2, 16, 16
NW = NC * NS
BPW = B // NW
CHUNK = 128
NCHUNK = BPW // CHUNK


_SPLIT_BLK = 2048
_SPLIT_GRID = (V + _SPLIT_BLK - 1) // _SPLIT_BLK


def _split_body(x_ref, *o_refs):
    for d in range(D):
        o_refs[d][...] = x_ref[d, :]


_split_tc = pl.pallas_call(
    _split_body,
    grid=(_SPLIT_GRID,),
    in_specs=[pl.BlockSpec((D, _SPLIT_BLK), lambda c: (0, c))],
    out_specs=[
        pl.BlockSpec((_SPLIT_BLK,), lambda c: (c,)) for _ in range(D)
    ],
    out_shape=[jax.ShapeDtypeStruct((V,), jnp.float32) for _ in range(D)],
)


_mesh = plsc.VectorSubcoreMesh(
    core_axis_name="c", subcore_axis_name="s", num_cores=NC, num_subcores=NS
)


@functools.partial(
    pl.kernel,
    out_type=tuple(
        jax.ShapeDtypeStruct((B,), jnp.float32) for _ in range(D)
    ),
    mesh=_mesh,
    compiler_params=pltpu.CompilerParams(
        needs_layout_passes=False, use_tc_tiling_on_sc=False
    ),
    scratch_types=[
        pltpu.VMEM((BPW,), jnp.int32),
        tuple(pltpu.VMEM((BPW,), jnp.float32) for _ in range(D)),
        pltpu.SemaphoreType.DMA,
        pltpu.SemaphoreType.DMA,
    ],
)
def _gather_sc(idx_hbm, *rest):
    tcol_hbm = rest[:D]
    out_refs = rest[D : 2 * D]
    idx_v, zd_vs, gsem, osem = rest[2 * D :]
    wid = lax.axis_index("s") * NC + lax.axis_index("c")
    base = wid * BPW

    pltpu.sync_copy(idx_hbm.at[pl.ds(base, BPW)], idx_v)
    copies = []
    for j in range(NCHUNK):
        sl = pl.ds(j * CHUNK, CHUNK)
        for d in range(D):
            copies.append(
                pltpu.async_copy(
                    tcol_hbm[d].at[idx_v.at[sl]], zd_vs[d].at[sl], gsem
                )
            )
    for c in copies:
        c.wait()

    outs = []
    for d in range(D):
        outs.append(
            pltpu.async_copy(zd_vs[d], out_refs[d].at[pl.ds(base, BPW)], osem)
        )
    for c in outs:
        c.wait()


_ASM_BLK = 512
_ASM_GRID = B // _ASM_BLK


def _assemble_body(g_ref, a_ref, o_ref, *zs_and_out):
    z_refs = zs_and_out[:D]
    out_ref = zs_and_out[D]
    out_ref[0, :] = g_ref[...].astype(jnp.float32)
    out_ref[1:8, :] = a_ref[...]
    out_ref[8:29, :] = o_ref[...]
    for d in range(D):
        out_ref[29 + d, :] = z_refs[d][...]


_assemble_tc = pl.pallas_call(
    _assemble_body,
    grid=(_ASM_GRID,),
    in_specs=(
        [pl.BlockSpec((_ASM_BLK,), lambda c: (c,))]
        + [pl.BlockSpec((7, _ASM_BLK), lambda c: (0, c))]
        + [pl.BlockSpec((21, _ASM_BLK), lambda c: (0, c))]
        + [pl.BlockSpec((_ASM_BLK,), lambda c: (c,)) for _ in range(D)]
    ),
    out_specs=pl.BlockSpec((NCOLS, _ASM_BLK), lambda c: (0, c)),
    out_shape=jax.ShapeDtypeStruct((NCOLS, B), jnp.float32),
)


def kernel(gender, age, occupation, zipcode_bucket, zipcode_table):
    tcols = _split_tc(jnp.swapaxes(zipcode_table, 0, 1))
    zcols = _gather_sc(zipcode_bucket.astype(jnp.int32), *tcols)
    out_t = _assemble_tc(
        gender.astype(jnp.int32),
        jnp.swapaxes(age, 0, 1),
        jnp.swapaxes(occupation, 0, 1),
        *zcols,
    )
    return jnp.swapaxes(out_t, 0, 1)
